# baseline (speedup 1.0000x reference)
import functools

import jax
import jax.numpy as jnp
from jax import lax
from jax.experimental import pallas as pl
from jax.experimental.pallas import tpu as pltpu
from jax.experimental.pallas import tpu_sc as plsc

N = 10000
E = 320000
NC = 2
NS = 16
NTILES = NC * NS
CH = 128
EPT = 10240
E_PAD = NTILES * EPT
NCH = EPT // CH
NP = 10240
RPT = NP // NS

_MESH = plsc.VectorSubcoreMesh(core_axis_name="c", subcore_axis_name="s")
_SC_PARAMS = pltpu.CompilerParams(use_tc_tiling_on_sc=False)


def _make_segsum(D, GRP):
  @functools.partial(
      pl.kernel,
      out_type=jax.ShapeDtypeStruct((NC, NP, D), jnp.float32),
      mesh=_MESH,
      compiler_params=_SC_PARAMS,
      scratch_types=(
          [pltpu.VMEM((NCH, CH), jnp.int32),
           pltpu.VMEM((NCH, CH), jnp.int32)] +
          [pltpu.VMEM((CH, D), jnp.float32) for _ in range(GRP)] +
          [pltpu.VMEM_SHARED((NP, D), jnp.float32),
           pltpu.SemaphoreType.DMA,
           pltpu.SemaphoreType.DMA]
      ),
  )
  def segsum(h_hbm, src_hbm, dst_hbm, zeros_hbm, out_hbm,
             src_v, dst_v, *rest):
    gbufs = rest[:GRP]
    acc, gsem, ssem = rest[GRP], rest[GRP + 1], rest[GRP + 2]
    cid = lax.axis_index("c")
    sid = lax.axis_index("s")
    wid = cid * NS + sid
    pltpu.sync_copy(src_hbm.at[wid], src_v)
    pltpu.sync_copy(dst_hbm.at[wid], dst_v)
    r0 = sid * RPT
    pltpu.sync_copy(zeros_hbm.at[pl.ds(r0, RPT)], acc.at[pl.ds(r0, RPT)])
    plsc.subcore_barrier()

    @pl.loop(0, NCH // GRP)
    def _(i):
      base = i * GRP
      for b in range(GRP):
        pltpu.sync_copy(h_hbm.at[src_v.at[base + b]], gbufs[b])
      for b in range(GRP):
        pltpu.sync_copy(gbufs[b], acc.at[dst_v.at[base + b]], add=True)

    plsc.subcore_barrier()
    pltpu.sync_copy(acc.at[pl.ds(r0, RPT)],
                    out_hbm.at[cid, pl.ds(r0, RPT)])

  return segsum


_segsum64 = _make_segsum(64, 4)
_segsum32 = _make_segsum(32, 8)

_DGRP = 8


@functools.partial(
    pl.kernel,
    out_type=(jax.ShapeDtypeStruct((NC, NP, 16), jnp.float32),
              jax.ShapeDtypeStruct((NC, NP, 16), jnp.float32)),
    mesh=_MESH,
    compiler_params=_SC_PARAMS,
    scratch_types=[
        pltpu.VMEM((NCH, CH), jnp.int32),
        pltpu.VMEM((NCH, CH), jnp.int32),
        pltpu.VMEM((CH, 16), jnp.float32),
        pltpu.VMEM_SHARED((NP, 16), jnp.float32),
        pltpu.VMEM_SHARED((NP, 16), jnp.float32),
        pltpu.SemaphoreType.DMA,
    ],
)
def _degrees(src_hbm, dst_hbm, ones_hbm, zeros_hbm, din_hbm, dout_hbm,
             src_v, dst_v, ones_v, acc_i, acc_o, sem):
  cid = lax.axis_index("c")
  sid = lax.axis_index("s")
  wid = cid * NS + sid
  pltpu.sync_copy(src_hbm.at[wid], src_v)
  pltpu.sync_copy(dst_hbm.at[wid], dst_v)
  pltpu.sync_copy(ones_hbm, ones_v)
  r0 = sid * RPT
  pltpu.sync_copy(zeros_hbm.at[pl.ds(r0, RPT)], acc_i.at[pl.ds(r0, RPT)])
  pltpu.sync_copy(zeros_hbm.at[pl.ds(r0, RPT)], acc_o.at[pl.ds(r0, RPT)])
  plsc.subcore_barrier()

  @pl.loop(0, NCH // _DGRP)
  def _(i):
    base = i * _DGRP
    cps = []
    for b in range(_DGRP):
      cps.append(pltpu.async_copy(ones_v, acc_i.at[dst_v.at[base + b]], sem,
                                  add=True))
      cps.append(pltpu.async_copy(ones_v, acc_o.at[src_v.at[base + b]], sem,
                                  add=True))
    for cp in cps:
      cp.wait()

  plsc.subcore_barrier()
  pltpu.sync_copy(acc_i.at[pl.ds(r0, RPT)], din_hbm.at[cid, pl.ds(r0, RPT)])
  pltpu.sync_copy(acc_o.at[pl.ds(r0, RPT)], dout_hbm.at[cid, pl.ds(r0, RPT)])


_R = 1000
_G = N // _R


def _leaky(v):
  return jnp.where(v >= 0.0, v, 0.01 * v)


def _tc1_body(din_ref, dout_ref, x_ref, h0n_ref, ni_ref, no_ref):
  deg_i = din_ref[0, :, 0] + din_ref[1, :, 0]
  deg_o = dout_ref[0, :, 0] + dout_ref[1, :, 0]
  ni = lax.rsqrt(jnp.maximum(deg_i, 1.0))
  no = lax.rsqrt(jnp.maximum(deg_o, 1.0))
  ni_ref[...] = ni[:, None]
  no_ref[...] = no[:, None]
  h0n_ref[...] = x_ref[...] * no[:, None]


def _tc1(din, dout, x):
  return pl.pallas_call(
      _tc1_body,
      grid=(_G,),
      in_specs=[
          pl.BlockSpec((NC, _R, 16), lambda i: (0, i, 0)),
          pl.BlockSpec((NC, _R, 16), lambda i: (0, i, 0)),
          pl.BlockSpec((_R, 128), lambda i: (i, 0)),
      ],
      out_specs=[
          pl.BlockSpec((_R, 128), lambda i: (i, 0)),
          pl.BlockSpec((_R, 1), lambda i: (i, 0)),
          pl.BlockSpec((_R, 1), lambda i: (i, 0)),
      ],
      out_shape=[
          jax.ShapeDtypeStruct((NP, 128), jnp.float32),
          jax.ShapeDtypeStruct((N, 1), jnp.float32),
          jax.ShapeDtypeStruct((N, 1), jnp.float32),
      ],
  )(din, dout, x)


def _tc2_body(agga_ref, aggb_ref, ni_ref, no_ref, w0_ref, w1_ref, out_ref):
  agga = agga_ref[0] + agga_ref[1]
  aggb = aggb_ref[0] + aggb_ref[1]
  w0 = w0_ref[...]
  t = (jnp.dot(agga, w0[:64], preferred_element_type=jnp.float32) +
       jnp.dot(aggb, w0[64:], preferred_element_type=jnp.float32))
  h = _leaky(t * ni_ref[...])
  hn = h * no_ref[...]
  out_ref[...] = jnp.dot(hn, w1_ref[...], preferred_element_type=jnp.float32)


def _tc2(agg0a, agg0b, ni, no, w0, w1):
  return pl.pallas_call(
      _tc2_body,
      grid=(_G,),
      in_specs=[
          pl.BlockSpec((NC, _R, 64), lambda i: (0, i, 0)),
          pl.BlockSpec((NC, _R, 64), lambda i: (0, i, 0)),
          pl.BlockSpec((_R, 1), lambda i: (i, 0)),
          pl.BlockSpec((_R, 1), lambda i: (i, 0)),
          pl.BlockSpec((128, 128), lambda i: (0, 0)),
          pl.BlockSpec((128, 64), lambda i: (0, 0)),
      ],
      out_specs=pl.BlockSpec((_R, 64), lambda i: (i, 0)),
      out_shape=jax.ShapeDtypeStruct((NP, 64), jnp.float32),
  )(agg0a, agg0b, ni, no, w0, w1)


def _tc3_body(agg_ref, ni_ref, no_ref, w2_ref, out_ref):
  agg = agg_ref[0] + agg_ref[1]
  h = _leaky(agg * ni_ref[...])
  hn = h * no_ref[...]
  out_ref[...] = jnp.dot(hn, w2_ref[...], preferred_element_type=jnp.float32)


def _tc3(agg1, ni, no, w2):
  return pl.pallas_call(
      _tc3_body,
      grid=(_G,),
      in_specs=[
          pl.BlockSpec((NC, _R, 64), lambda i: (0, i, 0)),
          pl.BlockSpec((_R, 1), lambda i: (i, 0)),
          pl.BlockSpec((_R, 1), lambda i: (i, 0)),
          pl.BlockSpec((64, 32), lambda i: (0, 0)),
      ],
      out_specs=pl.BlockSpec((_R, 32), lambda i: (i, 0)),
      out_shape=jax.ShapeDtypeStruct((NP, 32), jnp.float32),
  )(agg1, ni, no, w2)


def _tc4_body(agg_ref, ni_ref, wc_ref, out_ref):
  agg = agg_ref[0] + agg_ref[1]
  h = _leaky(agg * ni_ref[...])
  hg = jnp.sum(h, axis=0) * (1.0 / N)
  out_ref[...] = jnp.sum(wc_ref[...] * hg[None, :], axis=1)[None, :]


def _tc4(agg2, ni, wc):
  return pl.pallas_call(
      _tc4_body,
      grid=(1,),
      in_specs=[
          pl.BlockSpec((NC, N, 32), lambda i: (0, 0, 0)),
          pl.BlockSpec((N, 1), lambda i: (0, 0)),
          pl.BlockSpec((5, 32), lambda i: (0, 0)),
      ],
      out_specs=pl.BlockSpec((1, 5), lambda i: (0, 0)),
      out_shape=jax.ShapeDtypeStruct((1, 5), jnp.float32),
  )(agg2, ni, wc)


def kernel(x, edge_index, W0, W1, W2, Wc, bc):
  pad = jnp.full((E_PAD - E,), N, jnp.int32)
  src = jnp.concatenate([edge_index[0], pad]).reshape(NTILES, NCH, CH)
  dst = jnp.concatenate([edge_index[1], pad]).reshape(NTILES, NCH, CH)
  ones16 = jnp.ones((CH, 16), jnp.float32)
  z16 = jnp.zeros((NP, 16), jnp.float32)
  z64 = jnp.zeros((NP, 64), jnp.float32)
  z32 = jnp.zeros((NP, 32), jnp.float32)

  din, dout = _degrees(src, dst, ones16, z16)
  h0n, ni, no = _tc1(din, dout, x)
  agg0a = _segsum64(h0n[:, :64], src, dst, z64)
  agg0b = _segsum64(h0n[:, 64:], src, dst, z64)
  h1p = _tc2(agg0a, agg0b, ni, no, W0, W1)
  agg1 = _segsum64(h1p, src, dst, z64)
  h2p = _tc3(agg1, ni, no, W2)
  agg2 = _segsum32(h2p, src, dst, z32)
  out = _tc4(agg2, ni, Wc)
  return out.reshape(5) + bc


# speedup vs baseline: 2.6912x; 2.6912x over previous
import functools

import jax
import jax.numpy as jnp
from jax import lax
from jax.experimental import pallas as pl
from jax.experimental.pallas import tpu as pltpu
from jax.experimental.pallas import tpu_sc as plsc

N = 10000
E = 320000
NC = 2
NS = 16
NTILES = NC * NS
CH = 100
EPT = E // NTILES
NCH = EPT // CH
NP = 10240
RPT = NP // NS

_MESH = plsc.VectorSubcoreMesh(core_axis_name="c", subcore_axis_name="s")
_SC_PARAMS = pltpu.CompilerParams(use_tc_tiling_on_sc=False)


def _make_segsum(D, GRP):
  @functools.partial(
      pl.kernel,
      out_type=jax.ShapeDtypeStruct((NC, NP, D), jnp.float32),
      mesh=_MESH,
      compiler_params=_SC_PARAMS,
      scratch_types=(
          [pltpu.VMEM((NCH, CH), jnp.int32),
           pltpu.VMEM((NCH, CH), jnp.int32)] +
          [pltpu.VMEM((CH, D), jnp.float32) for _ in range(GRP)] +
          [pltpu.VMEM_SHARED((NP, D), jnp.float32),
           pltpu.SemaphoreType.DMA,
           pltpu.SemaphoreType.DMA]
      ),
  )
  def segsum(h_hbm, src_hbm, dst_hbm, zeros_hbm, out_hbm,
             src_v, dst_v, *rest):
    gbufs = rest[:GRP]
    acc, gsem, ssem = rest[GRP], rest[GRP + 1], rest[GRP + 2]
    cid = lax.axis_index("c")
    sid = lax.axis_index("s")
    wid = cid * NS + sid
    pltpu.sync_copy(src_hbm.at[wid], src_v)
    pltpu.sync_copy(dst_hbm.at[wid], dst_v)
    r0 = sid * RPT
    pltpu.sync_copy(zeros_hbm.at[pl.ds(r0, RPT)], acc.at[pl.ds(r0, RPT)])
    plsc.subcore_barrier()

    @pl.loop(0, NCH // GRP)
    def _(i):
      base = i * GRP
      gathers = [
          pltpu.async_copy(h_hbm.at[src_v.at[base + b]], gbufs[b], gsem)
          for b in range(GRP)
      ]
      for cp in gathers:
        cp.wait()
      scatters = [
          pltpu.async_copy(gbufs[b], acc.at[dst_v.at[base + b]], ssem,
                           add=True)
          for b in range(GRP)
      ]
      for cp in scatters:
        cp.wait()

    plsc.subcore_barrier()
    pltpu.sync_copy(acc.at[pl.ds(r0, RPT)],
                    out_hbm.at[cid, pl.ds(r0, RPT)])

  return segsum


_segsum128 = _make_segsum(128, 2)
_segsum64 = _make_segsum(64, 5)
_segsum32 = _make_segsum(32, 10)

_DGRP = 10


@functools.partial(
    pl.kernel,
    out_type=(jax.ShapeDtypeStruct((NC, NP, 16), jnp.float32),
              jax.ShapeDtypeStruct((NC, NP, 16), jnp.float32)),
    mesh=_MESH,
    compiler_params=_SC_PARAMS,
    scratch_types=[
        pltpu.VMEM((NCH, CH), jnp.int32),
        pltpu.VMEM((NCH, CH), jnp.int32),
        pltpu.VMEM((CH, 16), jnp.float32),
        pltpu.VMEM_SHARED((NP, 16), jnp.float32),
        pltpu.VMEM_SHARED((NP, 16), jnp.float32),
        pltpu.SemaphoreType.DMA,
    ],
)
def _degrees(src_hbm, dst_hbm, ones_hbm, zeros_hbm, din_hbm, dout_hbm,
             src_v, dst_v, ones_v, acc_i, acc_o, sem):
  cid = lax.axis_index("c")
  sid = lax.axis_index("s")
  wid = cid * NS + sid
  pltpu.sync_copy(src_hbm.at[wid], src_v)
  pltpu.sync_copy(dst_hbm.at[wid], dst_v)
  pltpu.sync_copy(ones_hbm, ones_v)
  r0 = sid * RPT
  pltpu.sync_copy(zeros_hbm.at[pl.ds(r0, RPT)], acc_i.at[pl.ds(r0, RPT)])
  pltpu.sync_copy(zeros_hbm.at[pl.ds(r0, RPT)], acc_o.at[pl.ds(r0, RPT)])
  plsc.subcore_barrier()

  @pl.loop(0, NCH // _DGRP)
  def _(i):
    base = i * _DGRP
    cps = []
    for b in range(_DGRP):
      cps.append(pltpu.async_copy(ones_v, acc_i.at[dst_v.at[base + b]], sem,
                                  add=True))
      cps.append(pltpu.async_copy(ones_v, acc_o.at[src_v.at[base + b]], sem,
                                  add=True))
    for cp in cps:
      cp.wait()

  plsc.subcore_barrier()
  pltpu.sync_copy(acc_i.at[pl.ds(r0, RPT)], din_hbm.at[cid, pl.ds(r0, RPT)])
  pltpu.sync_copy(acc_o.at[pl.ds(r0, RPT)], dout_hbm.at[cid, pl.ds(r0, RPT)])


_R = 1000
_G = N // _R


def _leaky(v):
  return jnp.where(v >= 0.0, v, 0.01 * v)


def _tc1_body(din_ref, dout_ref, x_ref, h0n_ref, ni_ref, no_ref):
  deg_i = din_ref[0, :, 0] + din_ref[1, :, 0]
  deg_o = dout_ref[0, :, 0] + dout_ref[1, :, 0]
  ni = lax.rsqrt(jnp.maximum(deg_i, 1.0))
  no = lax.rsqrt(jnp.maximum(deg_o, 1.0))
  ni_ref[...] = ni[:, None]
  no_ref[...] = no[:, None]
  h0n_ref[...] = x_ref[...] * no[:, None]


def _tc1(din, dout, x):
  return pl.pallas_call(
      _tc1_body,
      grid=(_G,),
      in_specs=[
          pl.BlockSpec((NC, _R, 16), lambda i: (0, i, 0)),
          pl.BlockSpec((NC, _R, 16), lambda i: (0, i, 0)),
          pl.BlockSpec((_R, 128), lambda i: (i, 0)),
      ],
      out_specs=[
          pl.BlockSpec((_R, 128), lambda i: (i, 0)),
          pl.BlockSpec((_R, 1), lambda i: (i, 0)),
          pl.BlockSpec((_R, 1), lambda i: (i, 0)),
      ],
      out_shape=[
          jax.ShapeDtypeStruct((NP, 128), jnp.float32),
          jax.ShapeDtypeStruct((N, 1), jnp.float32),
          jax.ShapeDtypeStruct((N, 1), jnp.float32),
      ],
  )(din, dout, x)


def _tc2_body(agg_ref, ni_ref, no_ref, w0_ref, w1_ref, out_ref):
  agg = agg_ref[0] + agg_ref[1]
  t = jnp.dot(agg, w0_ref[...], preferred_element_type=jnp.float32)
  h = _leaky(t * ni_ref[...])
  hn = h * no_ref[...]
  out_ref[...] = jnp.dot(hn, w1_ref[...], preferred_element_type=jnp.float32)


def _tc2(agg0, ni, no, w0, w1):
  return pl.pallas_call(
      _tc2_body,
      grid=(_G,),
      in_specs=[
          pl.BlockSpec((NC, _R, 128), lambda i: (0, i, 0)),
          pl.BlockSpec((_R, 1), lambda i: (i, 0)),
          pl.BlockSpec((_R, 1), lambda i: (i, 0)),
          pl.BlockSpec((128, 128), lambda i: (0, 0)),
          pl.BlockSpec((128, 64), lambda i: (0, 0)),
      ],
      out_specs=pl.BlockSpec((_R, 64), lambda i: (i, 0)),
      out_shape=jax.ShapeDtypeStruct((NP, 64), jnp.float32),
  )(agg0, ni, no, w0, w1)


def _tc3_body(agg_ref, ni_ref, no_ref, w2_ref, out_ref):
  agg = agg_ref[0] + agg_ref[1]
  h = _leaky(agg * ni_ref[...])
  hn = h * no_ref[...]
  out_ref[...] = jnp.dot(hn, w2_ref[...], preferred_element_type=jnp.float32)


def _tc3(agg1, ni, no, w2):
  return pl.pallas_call(
      _tc3_body,
      grid=(_G,),
      in_specs=[
          pl.BlockSpec((NC, _R, 64), lambda i: (0, i, 0)),
          pl.BlockSpec((_R, 1), lambda i: (i, 0)),
          pl.BlockSpec((_R, 1), lambda i: (i, 0)),
          pl.BlockSpec((64, 32), lambda i: (0, 0)),
      ],
      out_specs=pl.BlockSpec((_R, 32), lambda i: (i, 0)),
      out_shape=jax.ShapeDtypeStruct((NP, 32), jnp.float32),
  )(agg1, ni, no, w2)


def _tc4_body(agg_ref, ni_ref, wc_ref, out_ref):
  agg = agg_ref[0] + agg_ref[1]
  h = _leaky(agg * ni_ref[...])
  hg = jnp.sum(h, axis=0) * (1.0 / N)
  out_ref[...] = jnp.sum(wc_ref[...] * hg[None, :], axis=1)[None, :]


def _tc4(agg2, ni, wc):
  return pl.pallas_call(
      _tc4_body,
      grid=(1,),
      in_specs=[
          pl.BlockSpec((NC, N, 32), lambda i: (0, 0, 0)),
          pl.BlockSpec((N, 1), lambda i: (0, 0)),
          pl.BlockSpec((5, 32), lambda i: (0, 0)),
      ],
      out_specs=pl.BlockSpec((1, 5), lambda i: (0, 0)),
      out_shape=jax.ShapeDtypeStruct((1, 5), jnp.float32),
  )(agg2, ni, wc)


def kernel(x, edge_index, W0, W1, W2, Wc, bc):
  src = edge_index[0].reshape(NTILES, NCH, CH)
  dst = edge_index[1].reshape(NTILES, NCH, CH)
  ones16 = jnp.ones((CH, 16), jnp.float32)
  z16 = jnp.zeros((NP, 16), jnp.float32)
  z128 = jnp.zeros((NP, 128), jnp.float32)
  z64 = jnp.zeros((NP, 64), jnp.float32)
  z32 = jnp.zeros((NP, 32), jnp.float32)

  din, dout = _degrees(src, dst, ones16, z16)
  h0n, ni, no = _tc1(din, dout, x)
  agg0 = _segsum128(h0n, src, dst, z128)
  h1p = _tc2(agg0, ni, no, W0, W1)
  agg1 = _segsum64(h1p, src, dst, z64)
  h2p = _tc3(agg1, ni, no, W2)
  agg2 = _segsum32(h2p, src, dst, z32)
  out = _tc4(agg2, ni, Wc)
  return out.reshape(5) + bc


# speedup vs baseline: 2.9244x; 1.0866x over previous
import functools

import jax
import jax.numpy as jnp
from jax import lax
from jax.experimental import pallas as pl
from jax.experimental.pallas import tpu as pltpu
from jax.experimental.pallas import tpu_sc as plsc

N = 10000
E = 320000
NC = 2
NS = 16
NTILES = NC * NS
CH = 100
EPT = E // NTILES
NCH = EPT // CH
NP = 10240
RPT = NP // NS

_MESH = plsc.VectorSubcoreMesh(core_axis_name="c", subcore_axis_name="s")
_SC_PARAMS = pltpu.CompilerParams(use_tc_tiling_on_sc=False)


SEG = 5
NCS = NCH // SEG
K = 4


def _ring_pass(h_hbm, src_hbm, dst_hbm, wid, src_v, dst_v, gbufs, acc,
               gsems, ssems):
  @pl.loop(0, SEG)
  def _(s):
    pltpu.sync_copy(src_hbm.at[wid, pl.ds(s * NCS, NCS)], src_v)
    pltpu.sync_copy(dst_hbm.at[wid, pl.ds(s * NCS, NCS)], dst_v)
    gd = [None] * NCS
    sd = [None] * NCS
    for c in range(K):
      gd[c] = pltpu.async_copy(h_hbm.at[src_v.at[c]], gbufs[c], gsems[c])
    for c in range(NCS):
      b = c % K
      gd[c].wait()
      sd[c] = pltpu.async_copy(gbufs[b], acc.at[dst_v.at[c]], ssems[b],
                               add=True)
      if c + K < NCS:
        sd[c].wait()
        gd[c + K] = pltpu.async_copy(h_hbm.at[src_v.at[c + K]], gbufs[b],
                                     gsems[b])
    for c in range(NCS - K, NCS):
      sd[c].wait()


def _make_segsum(D):
  @functools.partial(
      pl.kernel,
      out_type=jax.ShapeDtypeStruct((NC, NP, D), jnp.float32),
      mesh=_MESH,
      compiler_params=_SC_PARAMS,
      scratch_types=(
          [pltpu.VMEM((NCS, CH), jnp.int32),
           pltpu.VMEM((NCS, CH), jnp.int32)] +
          [pltpu.VMEM((CH, D), jnp.float32) for _ in range(K)] +
          [pltpu.VMEM_SHARED((NP, D), jnp.float32)] +
          [pltpu.SemaphoreType.DMA] * (2 * K)
      ),
  )
  def segsum(h_hbm, src_hbm, dst_hbm, zeros_hbm, out_hbm,
             src_v, dst_v, *rest):
    gbufs = rest[:K]
    acc = rest[K]
    gsems = rest[K + 1:2 * K + 1]
    ssems = rest[2 * K + 1:3 * K + 1]
    cid = lax.axis_index("c")
    sid = lax.axis_index("s")
    wid = cid * NS + sid
    r0 = sid * RPT
    pltpu.sync_copy(zeros_hbm.at[pl.ds(r0, RPT)], acc.at[pl.ds(r0, RPT)])
    plsc.subcore_barrier()
    _ring_pass(h_hbm, src_hbm, dst_hbm, wid, src_v, dst_v, gbufs, acc,
               gsems, ssems)
    plsc.subcore_barrier()
    pltpu.sync_copy(acc.at[pl.ds(r0, RPT)],
                    out_hbm.at[cid, pl.ds(r0, RPT)])

  return segsum


@functools.partial(
    pl.kernel,
    out_type=jax.ShapeDtypeStruct((NC, 2, NP, 64), jnp.float32),
    mesh=_MESH,
    compiler_params=_SC_PARAMS,
    scratch_types=(
        [pltpu.VMEM((NCS, CH), jnp.int32),
         pltpu.VMEM((NCS, CH), jnp.int32)] +
        [pltpu.VMEM((CH, 64), jnp.float32) for _ in range(K)] +
        [pltpu.VMEM_SHARED((NP, 64), jnp.float32)] +
        [pltpu.SemaphoreType.DMA] * (2 * K)
    ),
)
def _segsum0(ha_hbm, hb_hbm, src_hbm, dst_hbm, zeros_hbm, out_hbm,
             src_v, dst_v, *rest):
  gbufs = rest[:K]
  acc = rest[K]
  gsems = rest[K + 1:2 * K + 1]
  ssems = rest[2 * K + 1:3 * K + 1]
  cid = lax.axis_index("c")
  sid = lax.axis_index("s")
  wid = cid * NS + sid
  r0 = sid * RPT
  for p, h_hbm in enumerate((ha_hbm, hb_hbm)):
    pltpu.sync_copy(zeros_hbm.at[pl.ds(r0, RPT)], acc.at[pl.ds(r0, RPT)])
    plsc.subcore_barrier()
    _ring_pass(h_hbm, src_hbm, dst_hbm, wid, src_v, dst_v, gbufs, acc,
               gsems, ssems)
    plsc.subcore_barrier()
    pltpu.sync_copy(acc.at[pl.ds(r0, RPT)],
                    out_hbm.at[cid, p, pl.ds(r0, RPT)])
    plsc.subcore_barrier()


_segsum64 = _make_segsum(64)
_segsum32 = _make_segsum(32)

_DGRP = 10


@functools.partial(
    pl.kernel,
    out_type=jax.ShapeDtypeStruct((NC, NP, 16), jnp.float32),
    mesh=_MESH,
    compiler_params=_SC_PARAMS,
    scratch_types=[
        pltpu.VMEM((NCH, CH), jnp.int32),
        pltpu.VMEM((NCH, CH), jnp.int32),
        pltpu.VMEM((CH, 16), jnp.float32),
        pltpu.VMEM((CH, 16), jnp.float32),
        pltpu.VMEM_SHARED((NP, 16), jnp.float32),
        pltpu.SemaphoreType.DMA,
    ],
)
def _degrees(src_hbm, dst_hbm, onesi_hbm, oneso_hbm, zeros_hbm, deg_hbm,
             src_v, dst_v, onesi_v, oneso_v, acc, sem):
  cid = lax.axis_index("c")
  sid = lax.axis_index("s")
  wid = cid * NS + sid
  pltpu.sync_copy(src_hbm.at[wid], src_v)
  pltpu.sync_copy(dst_hbm.at[wid], dst_v)
  pltpu.sync_copy(onesi_hbm, onesi_v)
  pltpu.sync_copy(oneso_hbm, oneso_v)
  r0 = sid * RPT
  pltpu.sync_copy(zeros_hbm.at[pl.ds(r0, RPT)], acc.at[pl.ds(r0, RPT)])
  plsc.subcore_barrier()

  @pl.loop(0, NCH // _DGRP)
  def _(i):
    base = i * _DGRP
    cps = []
    for b in range(_DGRP):
      cps.append(pltpu.async_copy(onesi_v, acc.at[dst_v.at[base + b]], sem,
                                  add=True))
      cps.append(pltpu.async_copy(oneso_v, acc.at[src_v.at[base + b]], sem,
                                  add=True))
    for cp in cps:
      cp.wait()

  plsc.subcore_barrier()
  pltpu.sync_copy(acc.at[pl.ds(r0, RPT)], deg_hbm.at[cid, pl.ds(r0, RPT)])


_R = 1000
_G = N // _R


def _leaky(v):
  return jnp.where(v >= 0.0, v, 0.01 * v)


def _tc1_body(deg_ref, x_ref, h0a_ref, h0b_ref, ni_ref, no_ref):
  deg_i = deg_ref[0, :, 0] + deg_ref[1, :, 0]
  deg_o = deg_ref[0, :, 8] + deg_ref[1, :, 8]
  ni = lax.rsqrt(jnp.maximum(deg_i, 1.0))
  no = lax.rsqrt(jnp.maximum(deg_o, 1.0))
  ni_ref[...] = ni[:, None]
  no_ref[...] = no[:, None]
  x = x_ref[...]
  h0a_ref[...] = x[:, :64] * no[:, None]
  h0b_ref[...] = x[:, 64:] * no[:, None]


def _tc1(deg, x):
  return pl.pallas_call(
      _tc1_body,
      grid=(_G,),
      in_specs=[
          pl.BlockSpec((NC, _R, 16), lambda i: (0, i, 0)),
          pl.BlockSpec((_R, 128), lambda i: (i, 0)),
      ],
      out_specs=[
          pl.BlockSpec((_R, 64), lambda i: (i, 0)),
          pl.BlockSpec((_R, 64), lambda i: (i, 0)),
          pl.BlockSpec((_R, 1), lambda i: (i, 0)),
          pl.BlockSpec((_R, 1), lambda i: (i, 0)),
      ],
      out_shape=[
          jax.ShapeDtypeStruct((NP, 64), jnp.float32),
          jax.ShapeDtypeStruct((NP, 64), jnp.float32),
          jax.ShapeDtypeStruct((N, 1), jnp.float32),
          jax.ShapeDtypeStruct((N, 1), jnp.float32),
      ],
  )(deg, x)


def _tc2_body(agg_ref, ni_ref, no_ref, w0_ref, w1_ref, out_ref):
  agga = agg_ref[0, 0] + agg_ref[1, 0]
  aggb = agg_ref[0, 1] + agg_ref[1, 1]
  w0 = w0_ref[...]
  t = (jnp.dot(agga, w0[:64], preferred_element_type=jnp.float32) +
       jnp.dot(aggb, w0[64:], preferred_element_type=jnp.float32))
  h = _leaky(t * ni_ref[...])
  hn = h * no_ref[...]
  out_ref[...] = jnp.dot(hn, w1_ref[...], preferred_element_type=jnp.float32)


def _tc2(agg0, ni, no, w0, w1):
  return pl.pallas_call(
      _tc2_body,
      grid=(_G,),
      in_specs=[
          pl.BlockSpec((NC, 2, _R, 64), lambda i: (0, 0, i, 0)),
          pl.BlockSpec((_R, 1), lambda i: (i, 0)),
          pl.BlockSpec((_R, 1), lambda i: (i, 0)),
          pl.BlockSpec((128, 128), lambda i: (0, 0)),
          pl.BlockSpec((128, 64), lambda i: (0, 0)),
      ],
      out_specs=pl.BlockSpec((_R, 64), lambda i: (i, 0)),
      out_shape=jax.ShapeDtypeStruct((NP, 64), jnp.float32),
  )(agg0, ni, no, w0, w1)


def _tc3_body(agg_ref, ni_ref, no_ref, w2_ref, out_ref):
  agg = agg_ref[0] + agg_ref[1]
  h = _leaky(agg * ni_ref[...])
  hn = h * no_ref[...]
  out_ref[...] = jnp.dot(hn, w2_ref[...], preferred_element_type=jnp.float32)


def _tc3(agg1, ni, no, w2):
  return pl.pallas_call(
      _tc3_body,
      grid=(_G,),
      in_specs=[
          pl.BlockSpec((NC, _R, 64), lambda i: (0, i, 0)),
          pl.BlockSpec((_R, 1), lambda i: (i, 0)),
          pl.BlockSpec((_R, 1), lambda i: (i, 0)),
          pl.BlockSpec((64, 32), lambda i: (0, 0)),
      ],
      out_specs=pl.BlockSpec((_R, 32), lambda i: (i, 0)),
      out_shape=jax.ShapeDtypeStruct((NP, 32), jnp.float32),
  )(agg1, ni, no, w2)


def _tc4_body(agg_ref, ni_ref, wc_ref, out_ref):
  agg = agg_ref[0] + agg_ref[1]
  h = _leaky(agg * ni_ref[...])
  hg = jnp.sum(h, axis=0) * (1.0 / N)
  out_ref[...] = jnp.sum(wc_ref[...] * hg[None, :], axis=1)[None, :]


def _tc4(agg2, ni, wc):
  return pl.pallas_call(
      _tc4_body,
      grid=(1,),
      in_specs=[
          pl.BlockSpec((NC, N, 32), lambda i: (0, 0, 0)),
          pl.BlockSpec((N, 1), lambda i: (0, 0)),
          pl.BlockSpec((5, 32), lambda i: (0, 0)),
      ],
      out_specs=pl.BlockSpec((1, 5), lambda i: (0, 0)),
      out_shape=jax.ShapeDtypeStruct((1, 5), jnp.float32),
  )(agg2, ni, wc)


def kernel(x, edge_index, W0, W1, W2, Wc, bc):
  src = edge_index[0].reshape(NTILES, NCH, CH)
  dst = edge_index[1].reshape(NTILES, NCH, CH)
  lane = jnp.arange(16)
  onesi = jnp.broadcast_to((lane < 8).astype(jnp.float32), (CH, 16))
  oneso = jnp.broadcast_to((lane >= 8).astype(jnp.float32), (CH, 16))
  z16 = jnp.zeros((NP, 16), jnp.float32)
  z64 = jnp.zeros((NP, 64), jnp.float32)
  z32 = jnp.zeros((NP, 32), jnp.float32)

  deg = _degrees(src, dst, onesi, oneso, z16)
  h0a, h0b, ni, no = _tc1(deg, x)
  agg0 = _segsum0(h0a, h0b, src, dst, z64)
  h1p = _tc2(agg0, ni, no, W0, W1)
  agg1 = _segsum64(h1p, src, dst, z64)
  h2p = _tc3(agg1, ni, no, W2)
  agg2 = _segsum32(h2p, src, dst, z32)
  out = _tc4(agg2, ni, Wc)
  return out.reshape(5) + bc
